# baseline (speedup 1.0000x reference)
import functools

import jax
import jax.numpy as jnp
from jax import lax
from jax.experimental import pallas as pl
from jax.experimental.pallas import tpu as pltpu
from jax.experimental.pallas import tpu_sc as plsc

N_USERS = 4096
N_POSTS = 4096
N_ALL = N_USERS + N_POSTS
H = 128

_SC_CORES = 2
_SC_SUBCORES = 16
_SC_WORKERS = _SC_CORES * _SC_SUBCORES
_GCHUNK = 128

_INTERPRET = False


_RM = 256
_NB = N_ALL // _RM


def _gnn_kernel(uf_ref, pf_ref, adj_ref, wu_ref, bu_ref, wp_ref, bp_ref,
                wg0_ref, bg0_ref, wg1_ref, bg1_ref, out_ref, s0_ref, s1_ref):
    f32 = jnp.float32
    i = pl.program_id(0)

    @pl.when(i == 0)
    def _proj():
        for h, (f_ref, w_ref, b_ref) in enumerate(
                ((uf_ref, wu_ref, bu_ref), (pf_ref, wp_ref, bp_ref))):
            emb = jnp.dot(f_ref[...], w_ref[...],
                          preferred_element_type=f32) + b_ref[...]
            s0_ref[pl.ds(h * N_USERS, N_USERS), :] = jnp.dot(
                emb, wg0_ref[...], preferred_element_type=f32)

    @pl.when((i >= 1) & (i <= _NB))
    def _l1():
        acc = jnp.dot(adj_ref[...], s0_ref[...], preferred_element_type=f32)
        h1 = jnp.maximum(acc + bg0_ref[...], 0.0)
        s1_ref[pl.ds((i - 1) * _RM, _RM), :] = jnp.dot(
            h1, wg1_ref[...], preferred_element_type=f32)

    @pl.when(i > _NB)
    def _l2():
        acc = jnp.dot(adj_ref[...], s1_ref[...], preferred_element_type=f32)
        out_ref[...] = jnp.maximum(acc + bg1_ref[...], 0.0)


def _gather_body(table_hbm, uidx_hbm, pidx_hbm, out_hbm,
                 idx_u, idx_p, rows_u, rows_p, sem_i, sem_g):
    wid = lax.axis_index("s") * _SC_CORES + lax.axis_index("c")
    base = wid * _GCHUNK
    cp_u = pltpu.async_copy(uidx_hbm.at[pl.ds(base, _GCHUNK)], idx_u, sem_i)
    cp_p = pltpu.async_copy(pidx_hbm.at[pl.ds(base, _GCHUNK)], idx_p, sem_i)
    cp_u.wait()
    g_u = pltpu.async_copy(table_hbm.at[idx_u], rows_u, sem_g)
    cp_p.wait()
    for k in range(_GCHUNK // 16):
        sl = pl.ds(k * 16, 16)
        idx_p[sl] = idx_p[sl] + N_USERS
    g_p = pltpu.async_copy(table_hbm.at[idx_p], rows_p, sem_g)
    g_u.wait()
    pltpu.sync_copy(rows_u, out_hbm.at[pl.ds(base, _GCHUNK)])
    g_p.wait()
    pltpu.sync_copy(rows_p, out_hbm.at[pl.ds(N_USERS + base, _GCHUNK)])


_gather_rows = functools.partial(
    pl.kernel,
    out_type=jax.ShapeDtypeStruct((2 * N_USERS, H), jnp.float32),
    scratch_types=[
        pltpu.VMEM((_GCHUNK,), jnp.int32),
        pltpu.VMEM((_GCHUNK,), jnp.int32),
        pltpu.VMEM((_GCHUNK, H), jnp.float32),
        pltpu.VMEM((_GCHUNK, H), jnp.float32),
        pltpu.SemaphoreType.DMA,
        pltpu.SemaphoreType.DMA,
    ],
    mesh=plsc.VectorSubcoreMesh(core_axis_name="c", subcore_axis_name="s"),
)(_gather_body)


def _head_kernel(bu_ref, bp_ref, w0u_ref, w0p_ref, b0_ref, w1_ref, b1_ref,
                 w2_ref, b2_ref, out_ref):
    x = (jnp.dot(bu_ref[...], w0u_ref[...], preferred_element_type=jnp.float32)
         + jnp.dot(bp_ref[...], w0p_ref[...], preferred_element_type=jnp.float32)
         + b0_ref[...])
    x = jnp.maximum(x, 0.0)
    x = jnp.maximum(
        jnp.dot(x, w1_ref[...], preferred_element_type=jnp.float32) + b1_ref[...],
        0.0)
    s = jnp.dot(x, w2_ref[...], preferred_element_type=jnp.float32) + b2_ref[...]
    out_ref[...] = jax.nn.sigmoid(s)


def kernel(user_features, post_features, adj_matrix, user_indices, post_indices,
           Wu, bu, Wp, bp, Wg0, bg0, Wg1, bg1, Wh0, bh0, Wh1, bh1, Wh2, bh2):
    f32 = jnp.float32
    d_in = user_features.shape[1]

    adj_map = lambda i: (jnp.where(i > _NB, i - _NB - 1, jnp.maximum(i - 1, 0)), 0)
    const2 = lambda i: (0, 0)
    h2 = pl.pallas_call(
        _gnn_kernel,
        grid=(2 * _NB + 1,),
        in_specs=[
            pl.BlockSpec((N_USERS, d_in), const2),
            pl.BlockSpec((N_POSTS, d_in), const2),
            pl.BlockSpec((_RM, N_ALL), adj_map),
            pl.BlockSpec((d_in, H), const2),
            pl.BlockSpec((1, H), const2),
            pl.BlockSpec((d_in, H), const2),
            pl.BlockSpec((1, H), const2),
            pl.BlockSpec((H, H), const2),
            pl.BlockSpec((1, H), const2),
            pl.BlockSpec((H, H), const2),
            pl.BlockSpec((1, H), const2),
        ],
        out_specs=pl.BlockSpec((_RM, H), lambda i: (jnp.maximum(i - _NB - 1, 0), 0)),
        out_shape=jax.ShapeDtypeStruct((N_ALL, H), f32),
        scratch_shapes=[
            pltpu.VMEM((N_ALL, H), f32),
            pltpu.VMEM((N_ALL, H), f32),
        ],
        interpret=_INTERPRET,
    )(user_features, post_features, adj_matrix, Wu, bu.reshape(1, H),
      Wp, bp.reshape(1, H), Wg0, bg0.reshape(1, H), Wg1, bg1.reshape(1, H))

    gathered = _gather_rows(
        h2, user_indices.astype(jnp.int32), post_indices.astype(jnp.int32))

    BB = 1024
    nb = N_USERS // BB
    scores = pl.pallas_call(
        _head_kernel,
        grid=(nb,),
        in_specs=[
            pl.BlockSpec((BB, H), lambda i: (i, 0)),
            pl.BlockSpec((BB, H), lambda i: (i + nb, 0)),
            pl.BlockSpec((H, H), lambda i: (0, 0)),
            pl.BlockSpec((H, H), lambda i: (0, 0)),
            pl.BlockSpec((1, H), lambda i: (0, 0)),
            pl.BlockSpec((H, H // 2), lambda i: (0, 0)),
            pl.BlockSpec((1, H // 2), lambda i: (0, 0)),
            pl.BlockSpec((H // 2, 1), lambda i: (0, 0)),
            pl.BlockSpec((1, 1), lambda i: (0, 0)),
        ],
        out_specs=pl.BlockSpec((BB, 1), lambda i: (i, 0)),
        out_shape=jax.ShapeDtypeStruct((N_USERS, 1), f32),
        interpret=_INTERPRET,
    )(gathered, gathered, Wh0[:H], Wh0[H:], bh0.reshape(1, H),
      Wh1, bh1.reshape(1, H // 2), Wh2, bh2.reshape(1, 1))
    return jnp.squeeze(scores, axis=-1)
